# Initial kernel scaffold; baseline (speedup 1.0000x reference)
import functools

import jax
import jax.numpy as jnp
from jax import lax
from jax.experimental import pallas as pl
from jax.experimental.pallas import tpu as pltpu
from jax.experimental.pallas import tpu_sc as plsc

N = 10000
E = 320000
D = 128
NP = 10240
NC = 2
NS = 16
NW = NC * NS
CHUNK = 128
CHUNKS = 79
E_PAD = NW * CHUNKS * CHUNK
ROWS_PER_SUB = NP // NS

_mesh = plsc.VectorSubcoreMesh(core_axis_name="c", subcore_axis_name="s")


@functools.partial(
    pl.kernel,
    out_type=jax.ShapeDtypeStruct((NW, NP), jnp.float32),
    mesh=_mesh,
    scratch_types=[
        pltpu.VMEM((CHUNKS * CHUNK,), jnp.int32),
        pltpu.VMEM((NP,), jnp.float32),
    ],
)
def _sc_degree_hist(dst_hbm, out_hbm, dstv, hist):
    wid = lax.axis_index("s") * NC + lax.axis_index("c")
    pltpu.sync_copy(dst_hbm.at[wid], dstv)

    @pl.loop(0, NP, step=16)
    def _(i):
        hist[pl.ds(i, 16)] = jnp.zeros((16,), jnp.float32)

    ones = jnp.ones((16,), jnp.float32)

    @pl.loop(0, CHUNKS * CHUNK, step=16)
    def _(i):
        idx = dstv[pl.ds(i, 16)]
        plsc.addupdate_scatter(hist, [idx], ones)

    pltpu.sync_copy(hist, out_hbm.at[wid])


@functools.partial(
    pl.kernel,
    out_type=jax.ShapeDtypeStruct((NC, NP, D), jnp.float32),
    mesh=_mesh,
    scratch_types=[
        pltpu.VMEM((CHUNKS, CHUNK), jnp.int32),
        pltpu.VMEM((CHUNKS, CHUNK), jnp.int32),
        pltpu.VMEM((CHUNK, D), jnp.float32),
        pltpu.VMEM_SHARED((NP, D), jnp.float32),
        pltpu.SemaphoreType.DMA,
    ],
)
def _sc_scatter_accum(g_hbm, srci_hbm, dsti_hbm, out_hbm, srcv, dstv, buf, acc, sem):
    c = lax.axis_index("c")
    s = lax.axis_index("s")
    wid = s * NC + c
    pltpu.sync_copy(srci_hbm.at[wid], srcv)
    pltpu.sync_copy(dsti_hbm.at[wid], dstv)

    @pl.loop(0, CHUNK)
    def _(r):
        @pl.loop(0, D, step=16)
        def _(cc):
            buf[r, pl.ds(cc, 16)] = jnp.zeros((16,), jnp.float32)

    @pl.loop(0, ROWS_PER_SUB, step=CHUNK)
    def _(r):
        pltpu.sync_copy(buf, acc.at[pl.ds(s * ROWS_PER_SUB + r, CHUNK)])

    plsc.subcore_barrier()

    @pl.loop(0, CHUNKS)
    def _(j):
        pltpu.async_copy(g_hbm.at[srcv.at[j]], buf, sem).wait()
        pltpu.sync_copy(buf, acc.at[dstv.at[j]], add=True)

    plsc.subcore_barrier()
    pltpu.sync_copy(
        acc.at[pl.ds(s * ROWS_PER_SUB, ROWS_PER_SUB)],
        out_hbm.at[c, pl.ds(s * ROWS_PER_SUB, ROWS_PER_SUB)],
    )


_BLK1 = 1280


def _tc_scale_body(degp_ref, h_ref, g_ref, dinv_ref):
    deg = jnp.sum(degp_ref[...], axis=0) + 1.0
    dinv = lax.rsqrt(deg)
    dinv_ref[...] = dinv
    g_ref[...] = h_ref[...] * dinv[:, None]


def _tc_scale(deg_part, h_pad):
    return pl.pallas_call(
        _tc_scale_body,
        grid=(NP // _BLK1,),
        in_specs=[
            pl.BlockSpec((NW, _BLK1), lambda i: (0, i)),
            pl.BlockSpec((_BLK1, D), lambda i: (i, 0)),
        ],
        out_specs=[
            pl.BlockSpec((_BLK1, D), lambda i: (i, 0)),
            pl.BlockSpec((_BLK1,), lambda i: (i,)),
        ],
        out_shape=[
            jax.ShapeDtypeStruct((NP, D), jnp.float32),
            jax.ShapeDtypeStruct((NP,), jnp.float32),
        ],
    )(deg_part, h_pad)


_BLK2 = 2000


def _tc_combine_body(s0_ref, s1_ref, g_ref, dinv_ref, w_ref, b_ref, out_ref):
    agg = s0_ref[...] + s1_ref[...] + g_ref[...]
    agg = agg * dinv_ref[...][:, None]
    out_ref[...] = (
        lax.dot_general(
            agg,
            w_ref[...],
            (((1,), (1,)), ((), ())),
            precision=lax.Precision.HIGHEST,
            preferred_element_type=jnp.float32,
        )
        + b_ref[...][None, :]
    )


def _tc_combine(s0, s1, g, dinv, w, b):
    return pl.pallas_call(
        _tc_combine_body,
        grid=(N // _BLK2,),
        in_specs=[
            pl.BlockSpec((_BLK2, D), lambda i: (i, 0)),
            pl.BlockSpec((_BLK2, D), lambda i: (i, 0)),
            pl.BlockSpec((_BLK2, D), lambda i: (i, 0)),
            pl.BlockSpec((_BLK2,), lambda i: (i,)),
            pl.BlockSpec((D, D), lambda i: (0, 0)),
            pl.BlockSpec((D,), lambda i: (0,)),
        ],
        out_specs=pl.BlockSpec((_BLK2, D), lambda i: (i, 0)),
        out_shape=jax.ShapeDtypeStruct((N, D), jnp.float32),
    )(s0, s1, g, dinv, w, b)


def kernel(H, edge_index, W, b):
    src = edge_index[0]
    dst = edge_index[1]
    pad = jnp.full((E_PAD - E,), N, jnp.int32)
    src_p = jnp.concatenate([src, pad]).reshape(NW, CHUNKS, CHUNK)
    dst_p = jnp.concatenate([dst, pad]).reshape(NW, CHUNKS, CHUNK)
    dst_flat = dst_p.reshape(NW, CHUNKS * CHUNK)
    h_pad = jnp.pad(H, ((0, NP - N), (0, 0)))

    deg_part = _sc_degree_hist(dst_flat)
    g, dinv = _tc_scale(deg_part, h_pad)
    s_part = _sc_scatter_accum(g, src_p, dst_p)
    return _tc_combine(s_part[0], s_part[1], g, dinv, W, b)


# speedup vs baseline: 19.1811x; 19.1811x over previous
import dataclasses
import functools

import jax
import jax.numpy as jnp
from jax import lax
from jax.experimental import pallas as pl
from jax.experimental.pallas import tpu as pltpu
from jax.experimental.pallas import tpu_sc as plsc

N = 10000
E = 320000
D = 128
NP = 10240
NC = 2
NS = 16
NW = NC * NS
CHUNK = 128
CHUNKS = 79
E_PAD = NW * CHUNKS * CHUNK
ROWS_PER_SUB = NP // NS

_mesh = plsc.VectorSubcoreMesh(core_axis_name="c", subcore_axis_name="s")

_cp = pltpu.CompilerParams()
if "needs_layout_passes" in pltpu.CompilerParams.__dataclass_fields__:
    _cp = dataclasses.replace(_cp, needs_layout_passes=False)


@functools.partial(
    pl.kernel,
    out_type=jax.ShapeDtypeStruct((NW, NP), jnp.float32),
    mesh=_mesh,
    scratch_types=[
        pltpu.VMEM((CHUNKS * CHUNK,), jnp.int32),
        pltpu.VMEM((NP,), jnp.float32),
    ],
    compiler_params=_cp,
)
def _sc_degree_hist(dst_hbm, out_hbm, dstv, hist):
    wid = lax.axis_index("s") * NC + lax.axis_index("c")
    pltpu.sync_copy(dst_hbm.at[wid], dstv)

    @pl.loop(0, NP, step=16)
    def _(i):
        hist[pl.ds(i, 16)] = jnp.zeros((16,), jnp.float32)

    ones = jnp.ones((16,), jnp.float32)

    @pl.loop(0, CHUNKS * CHUNK, step=16)
    def _(i):
        idx = dstv[pl.ds(i, 16)]
        plsc.addupdate_scatter(hist, [idx], ones)

    pltpu.sync_copy(hist, out_hbm.at[wid])


@functools.partial(
    pl.kernel,
    out_type=jax.ShapeDtypeStruct((NC, NP, D), jnp.float32),
    mesh=_mesh,
    scratch_types=[
        pltpu.VMEM((CHUNKS, CHUNK), jnp.int32),
        pltpu.VMEM((CHUNKS, CHUNK), jnp.int32),
        pltpu.VMEM((CHUNK, D), jnp.float32),
        pltpu.VMEM_SHARED((NP, D), jnp.float32),
        pltpu.SemaphoreType.DMA,
    ],
)
def _sc_scatter_accum(g_hbm, srci_hbm, dsti_hbm, out_hbm, srcv, dstv, buf, acc, sem):
    c = lax.axis_index("c")
    s = lax.axis_index("s")
    wid = s * NC + c
    pltpu.sync_copy(srci_hbm.at[wid], srcv)
    pltpu.sync_copy(dsti_hbm.at[wid], dstv)

    @pl.loop(0, CHUNK)
    def _(r):
        @pl.loop(0, D, step=16)
        def _(cc):
            buf[r, pl.ds(cc, 16)] = jnp.zeros((16,), jnp.float32)

    @pl.loop(0, ROWS_PER_SUB, step=CHUNK)
    def _(r):
        pltpu.sync_copy(buf, acc.at[pl.ds(s * ROWS_PER_SUB + r, CHUNK)])

    plsc.subcore_barrier()

    @pl.loop(0, CHUNKS)
    def _(j):
        pltpu.async_copy(g_hbm.at[srcv.at[j]], buf, sem).wait()
        pltpu.sync_copy(buf, acc.at[dstv.at[j]], add=True)

    plsc.subcore_barrier()
    pltpu.sync_copy(
        acc.at[pl.ds(s * ROWS_PER_SUB, ROWS_PER_SUB)],
        out_hbm.at[c, pl.ds(s * ROWS_PER_SUB, ROWS_PER_SUB)],
    )


_BLK1 = 1280


def _tc_scale_body(degp_ref, h_ref, g_ref, dinv_ref):
    deg = jnp.sum(degp_ref[...], axis=0) + 1.0
    dinv = lax.rsqrt(deg)[:, None]
    dinv_ref[...] = dinv
    g_ref[...] = h_ref[...] * dinv


def _tc_scale(deg_part, h_pad):
    return pl.pallas_call(
        _tc_scale_body,
        grid=(NP // _BLK1,),
        in_specs=[
            pl.BlockSpec((NW, _BLK1), lambda i: (0, i)),
            pl.BlockSpec((_BLK1, D), lambda i: (i, 0)),
        ],
        out_specs=[
            pl.BlockSpec((_BLK1, D), lambda i: (i, 0)),
            pl.BlockSpec((_BLK1, 1), lambda i: (i, 0)),
        ],
        out_shape=[
            jax.ShapeDtypeStruct((NP, D), jnp.float32),
            jax.ShapeDtypeStruct((NP, 1), jnp.float32),
        ],
    )(deg_part, h_pad)


_BLK2 = 2000


def _tc_combine_body(s0_ref, s1_ref, g_ref, dinv_ref, w_ref, b_ref, out_ref):
    agg = s0_ref[...] + s1_ref[...] + g_ref[...]
    agg = agg * dinv_ref[...]
    out_ref[...] = (
        lax.dot_general(
            agg,
            w_ref[...],
            (((1,), (1,)), ((), ())),
            precision=lax.Precision.HIGHEST,
            preferred_element_type=jnp.float32,
        )
        + b_ref[...][None, :]
    )


def _tc_combine(s0, s1, g, dinv, w, b):
    return pl.pallas_call(
        _tc_combine_body,
        grid=(N // _BLK2,),
        in_specs=[
            pl.BlockSpec((_BLK2, D), lambda i: (i, 0)),
            pl.BlockSpec((_BLK2, D), lambda i: (i, 0)),
            pl.BlockSpec((_BLK2, D), lambda i: (i, 0)),
            pl.BlockSpec((_BLK2, 1), lambda i: (i, 0)),
            pl.BlockSpec((D, D), lambda i: (0, 0)),
            pl.BlockSpec((D,), lambda i: (0,)),
        ],
        out_specs=pl.BlockSpec((_BLK2, D), lambda i: (i, 0)),
        out_shape=jax.ShapeDtypeStruct((N, D), jnp.float32),
    )(s0, s1, g, dinv, w, b)


def kernel(H, edge_index, W, b):
    src = edge_index[0]
    dst = edge_index[1]
    pad = jnp.full((E_PAD - E,), N, jnp.int32)
    src_p = jnp.concatenate([src, pad]).reshape(NW, CHUNKS, CHUNK)
    dst_p = jnp.concatenate([dst, pad]).reshape(NW, CHUNKS, CHUNK)
    dst_flat = dst_p.reshape(NW, CHUNKS * CHUNK)
    h_pad = jnp.pad(H, ((0, NP - N), (0, 0)))

    deg_part = _sc_degree_hist(dst_flat)
    g, dinv = _tc_scale(deg_part, h_pad)
    s_part = _sc_scatter_accum(g, src_p, dst_p)
    return _tc_combine(s_part[0], s_part[1], g, dinv, W, b)
